# baseline (speedup 1.0000x reference)
import functools

import jax
import jax.numpy as jnp
from jax import lax
from jax.experimental import pallas as pl
from jax.experimental.pallas import tpu as pltpu
from jax.experimental.pallas import tpu_sc as plsc


def kernel(sequence, positions):
    B, S, D = sequence.shape
    _, N = positions.shape

    seq2d = sequence.reshape(B * S, D)
    npad = N + 4
    flat_idx = (
        jnp.pad(positions.astype(jnp.int32), ((0, 0), (0, npad - N)))
        + (jnp.arange(B, dtype=jnp.int32) * S)[:, None]
    ).reshape(B * npad)

    NC = 2
    rows = 40

    mesh = plsc.VectorSubcoreMesh(core_axis_name="c", subcore_axis_name="s")

    @functools.partial(
        pl.kernel,
        mesh=mesh,
        out_type=jax.ShapeDtypeStruct((B, N, D), jnp.float32),
        scratch_types=[
            pltpu.VMEM((rows,), jnp.int32),
            pltpu.VMEM((rows, D), jnp.float32),
            pltpu.VMEM((8, D), jnp.float32),
            pltpu.SemaphoreType.DMA,
        ],
    )
    def gather_kernel(table_hbm, idx_hbm, out_hbm, idx_v, rows_v, tail_v, sem):
        wid = lax.axis_index("s") * NC + lax.axis_index("c")
        b = wid // 8
        j = wid % 8

        @pl.when(j < 7)
        def _():
            base = pl.multiple_of(b * npad + j * rows, 8)
            start = pl.multiple_of(j * rows, 8)
            pltpu.sync_copy(idx_hbm.at[pl.ds(base, rows)], idx_v)
            pltpu.async_copy(table_hbm.at[idx_v], rows_v, sem).wait()
            pltpu.sync_copy(rows_v, out_hbm.at[b, pl.ds(start, rows)])

        @pl.when(j == 7)
        def _():
            base = pl.multiple_of(b * npad + 7 * rows, 8)
            pltpu.sync_copy(idx_hbm.at[pl.ds(base, 16)],
                            idx_v.at[pl.ds(0, 16)])
            pltpu.async_copy(table_hbm.at[idx_v.at[pl.ds(0, 16)]],
                             rows_v.at[pl.ds(0, 16)], sem).wait()
            pltpu.sync_copy(rows_v.at[pl.ds(0, 16)],
                            out_hbm.at[b, pl.ds(7 * rows, 16)])
            base2 = pl.multiple_of(b * npad + 7 * rows + 16, 8)
            pltpu.sync_copy(idx_hbm.at[pl.ds(base2, 8)],
                            idx_v.at[pl.ds(16, 8)])
            pltpu.async_copy(table_hbm.at[idx_v.at[pl.ds(16, 8)]],
                             tail_v, sem).wait()
            pltpu.sync_copy(tail_v.at[pl.ds(0, 4)],
                            out_hbm.at[b, pl.ds(N - 4, 4)])

    return gather_kernel(seq2d, flat_idx)


# speedup vs baseline: 1.7762x; 1.7762x over previous
import jax
import jax.numpy as jnp
from jax.experimental import pallas as pl
from jax.experimental.pallas import tpu as pltpu


def kernel(sequence, positions):
    B, S, D = sequence.shape
    _, N = positions.shape
    pos = positions.astype(jnp.int32)

    def body(idx_ref, seq_ref, out_ref, sem):
        b = pl.program_id(0)
        copies = []
        for r in range(N):
            idx = idx_ref[b, r]
            cp = pltpu.make_async_copy(
                seq_ref.at[b, pl.ds(idx, 1)],
                out_ref.at[0, pl.ds(r, 1)],
                sem,
            )
            cp.start()
            copies.append(cp)
        for cp in copies:
            cp.wait()

    return pl.pallas_call(
        body,
        grid_spec=pltpu.PrefetchScalarGridSpec(
            num_scalar_prefetch=1,
            grid=(B,),
            in_specs=[pl.BlockSpec(memory_space=pl.ANY)],
            out_specs=pl.BlockSpec((1, N, D), lambda b, idx_ref: (b, 0, 0)),
            scratch_shapes=[pltpu.SemaphoreType.DMA],
        ),
        out_shape=jax.ShapeDtypeStruct((B, N, D), jnp.float32),
    )(pos, sequence)
